# Initial kernel scaffold; baseline (speedup 1.0000x reference)
import jax
import jax.numpy as jnp
from jax.experimental import pallas as pl


def kernel(x, edge_index, cache_name, W1, b1, W2, b2):
    raise NotImplementedError("write your pallas kernel here")


# speedup vs baseline: 5.8840x; 5.8840x over previous
import functools

import jax
import jax.numpy as jnp
from jax import lax
from jax.experimental import pallas as pl
from jax.experimental.pallas import tpu as pltpu
from jax.experimental.pallas import tpu_sc as plsc

N = 10000
E = 160000
NT = 16
NC = 2
INIT_TILES = 10
INIT_ROWS = N // INIT_TILES


def _sc_mesh():
    return plsc.VectorSubcoreMesh(core_axis_name="c", subcore_axis_name="s",
                                  num_cores=NC, num_subcores=NT)


_DEG_E_TILE = E // (NC * NT)
_DEG_STAGE = _DEG_E_TILE + 16
_DEG_VREGS = -(-_DEG_E_TILE // 16)


@functools.cache
def _make_deg_kernel():
    return functools.partial(
        pl.kernel,
        out_type=jax.ShapeDtypeStruct((NC * NT * N,), jnp.float32),
        mesh=_sc_mesh(),
        scratch_types=[
            pltpu.VMEM((_DEG_STAGE,), jnp.int32),
            pltpu.VMEM((N,), jnp.float32),
        ],
        compiler_params=pltpu.CompilerParams(needs_layout_passes=False),
    )(_deg_body)


def _deg_body(dstp_hbm, zeros_hbm, out_hbm, ebuf, dloc):
    c = lax.axis_index("c")
    s = lax.axis_index("s")
    wid = c * NT + s
    lane = lax.iota(jnp.int32, 16)

    pltpu.sync_copy(zeros_hbm, dloc)
    tile_base = pl.multiple_of(wid * _DEG_E_TILE, 8)
    pltpu.sync_copy(dstp_hbm.at[pl.ds(tile_base, _DEG_STAGE)], ebuf)

    def body(j, carry):
        idx = ebuf[pl.ds(j * 16, 16)]
        valid = lane < jnp.minimum(_DEG_E_TILE - j * 16, 16)
        cnt, last = plsc.scan_count(idx, mask=valid)
        plsc.addupdate_scatter(dloc, (idx,), cnt.astype(jnp.float32),
                               mask=last)
        return carry

    lax.fori_loop(0, _DEG_VREGS, body, 0)
    out_base = pl.multiple_of(wid * N, 8)
    pltpu.sync_copy(dloc, out_hbm.at[pl.ds(out_base, N)])


def _prop_tile_loop(g_hbm, src_hbm, dst_hbm, acc, src_v, dst_v, rows_v, sem,
                    tile_base, num_batches, eb):
    def body(i, carry):
        base = pl.multiple_of(tile_base + i * eb, 8)
        pltpu.sync_copy(src_hbm.at[pl.ds(base, eb)], src_v)
        pltpu.async_copy(g_hbm.at[src_v], rows_v, sem).wait()
        pltpu.sync_copy(dst_hbm.at[pl.ds(base, eb)], dst_v)
        pltpu.sync_copy(rows_v, acc.at[dst_v], add=True)
        return carry

    lax.fori_loop(0, num_batches, body, 0)


def _spmem_init(src_hbm_rows, acc, s):
    @pl.when(s < INIT_TILES)
    def _():
        row0 = s * INIT_ROWS
        pltpu.sync_copy(src_hbm_rows.at[pl.ds(row0, INIT_ROWS)],
                        acc.at[pl.ds(row0, INIT_ROWS)])


def _spmem_writeout(acc, out_hbm, s):
    @pl.when(s < INIT_TILES)
    def _():
        row0 = s * INIT_ROWS
        pltpu.sync_copy(acc.at[pl.ds(row0, INIT_ROWS)],
                        out_hbm.at[pl.ds(row0, INIT_ROWS)])


EB1 = 40
EB2 = 80


@functools.cache
def _make_prop_edge(dh):
    @functools.partial(
        pl.kernel,
        out_type=(
            jax.ShapeDtypeStruct((N, dh), jnp.float32),
            jax.ShapeDtypeStruct((N, dh), jnp.float32),
        ),
        mesh=_sc_mesh(),
        scratch_types=[
            pltpu.VMEM((EB1,), jnp.int32),
            pltpu.VMEM((EB1,), jnp.int32),
            pltpu.VMEM((EB1, dh), jnp.float32),
            pltpu.VMEM_SHARED((N, dh), jnp.float32),
            pltpu.SemaphoreType.DMA,
        ],
        compiler_params=pltpu.CompilerParams(needs_layout_passes=False),
    )
    def prop(g_hbm, zeros_hbm, src_hbm, dst_hbm, out0, out1,
             src_v, dst_v, rows_v, acc, sem):
        c = lax.axis_index("c")
        s = lax.axis_index("s")
        e_per_tile = E // (NC * NT)

        def run(init_hbm, out_hbm, core_base):
            _spmem_init(init_hbm, acc, s)
            plsc.subcore_barrier()
            _prop_tile_loop(g_hbm, src_hbm, dst_hbm, acc,
                            src_v, dst_v, rows_v, sem,
                            core_base + s * e_per_tile,
                            e_per_tile // EB1, EB1)
            plsc.subcore_barrier()
            _spmem_writeout(acc, out_hbm, s)

        @pl.when(c == 0)
        def _():
            run(g_hbm, out0, 0)

        @pl.when(c == 1)
        def _():
            run(zeros_hbm, out1, E // 2)

    return prop


@functools.cache
def _make_prop_feat(dh):
    @functools.partial(
        pl.kernel,
        out_type=(
            jax.ShapeDtypeStruct((N, dh), jnp.float32),
            jax.ShapeDtypeStruct((N, dh), jnp.float32),
        ),
        mesh=_sc_mesh(),
        scratch_types=[
            pltpu.VMEM((EB2,), jnp.int32),
            pltpu.VMEM((EB2,), jnp.int32),
            pltpu.VMEM((EB2, dh), jnp.float32),
            pltpu.VMEM_SHARED((N, dh), jnp.float32),
            pltpu.SemaphoreType.DMA,
        ],
        compiler_params=pltpu.CompilerParams(needs_layout_passes=False),
    )
    def prop(g0_hbm, g1_hbm, src_hbm, dst_hbm, out0, out1,
             src_v, dst_v, rows_v, acc, sem):
        c = lax.axis_index("c")
        s = lax.axis_index("s")
        e_per_tile = E // NT

        def run(g_hbm, out_hbm):
            _spmem_init(g_hbm, acc, s)
            plsc.subcore_barrier()
            _prop_tile_loop(g_hbm, src_hbm, dst_hbm, acc,
                            src_v, dst_v, rows_v, sem,
                            s * e_per_tile, e_per_tile // EB2, EB2)
            plsc.subcore_barrier()
            _spmem_writeout(acc, out_hbm, s)

        @pl.when(c == 0)
        def _():
            run(g0_hbm, out0)

        @pl.when(c == 1)
        def _():
            run(g1_hbm, out1)

    return prop


_RB = 1000


def _dinv(degp_ref):
    deg = 1.0 + jnp.sum(degp_ref[:, :, 0], axis=0)
    return lax.rsqrt(deg)


def _k1_body(degp_ref, x_ref, w1_ref, g_ref):
    dinv = _dinv(degp_ref)
    hw = jnp.dot(x_ref[:, :], w1_ref[:, :], preferred_element_type=jnp.float32)
    g_ref[:, :] = hw * dinv[:, None]


def _k2_body(degp_ref, sa_ref, sb_ref, b1_ref, w2_ref,
             g0_ref, g1_ref):
    dinv = _dinv(degp_ref)
    sfull = sa_ref[:, :] + sb_ref[:, :]
    h1 = jax.nn.relu(sfull * dinv[:, None] + b1_ref[0, :][None, :])
    g = jnp.dot(h1, w2_ref[:, :], preferred_element_type=jnp.float32)
    g = g * dinv[:, None]
    h = g.shape[1] // 2
    g0_ref[:, :] = g[:, :h]
    g1_ref[:, :] = g[:, h:]


def _k3_body(degp_ref, s0_ref, s1_ref, b2_ref, out_ref):
    dinv = _dinv(degp_ref)
    sfull = jnp.concatenate([s0_ref[:, :], s1_ref[:, :]], axis=1)
    out_ref[:, :] = sfull * dinv[:, None] + b2_ref[0, :][None, :]


def _row_spec(w):
    return pl.BlockSpec((_RB, w), lambda i: (i, 0))


def _degp_spec():
    return pl.BlockSpec((NC * NT, _RB, 1), lambda i: (0, i, 0))


def _full_spec(shape):
    return pl.BlockSpec(shape, lambda i: tuple(0 for _ in shape))


def kernel(x, edge_index, cache_name, W1, b1, W2, b2):
    del cache_name
    n, d_in = x.shape
    d_hid = W1.shape[1]
    d_out = W2.shape[1]
    assert n == N and edge_index.shape == (2, E)
    grid = (n // _RB,)

    src = edge_index[0].astype(jnp.int32)
    dst = edge_index[1].astype(jnp.int32)
    dst_pad = jnp.concatenate([dst, jnp.zeros((16,), jnp.int32)])
    zeros_nh = jnp.zeros((N, d_hid), jnp.float32)

    zeros_flat = jnp.zeros((N,), jnp.float32)
    degp = _make_deg_kernel()(dst_pad, zeros_flat)
    degp = degp.reshape(NC * NT, N, 1)

    g1 = pl.pallas_call(
        _k1_body,
        grid=grid,
        in_specs=[_degp_spec(), _row_spec(d_in), _full_spec(W1.shape)],
        out_specs=_row_spec(d_hid),
        out_shape=jax.ShapeDtypeStruct((n, d_hid), jnp.float32),
    )(degp, x, W1)

    s1a, s1b = _make_prop_edge(d_hid)(g1, zeros_nh, src, dst)

    g20, g21 = pl.pallas_call(
        _k2_body,
        grid=grid,
        in_specs=[_degp_spec(), _row_spec(d_hid), _row_spec(d_hid),
                  _full_spec((1, d_hid)), _full_spec(W2.shape)],
        out_specs=[_row_spec(d_out // 2), _row_spec(d_out // 2)],
        out_shape=[jax.ShapeDtypeStruct((n, d_out // 2), jnp.float32)] * 2,
    )(degp, s1a, s1b, b1.reshape(1, d_hid), W2)

    s20, s21 = _make_prop_feat(d_out // 2)(g20, g21, src, dst)

    out = pl.pallas_call(
        _k3_body,
        grid=grid,
        in_specs=[_degp_spec(), _row_spec(d_out // 2), _row_spec(d_out // 2),
                  _full_spec((1, d_out))],
        out_specs=_row_spec(d_out),
        out_shape=jax.ShapeDtypeStruct((n, d_out), jnp.float32),
    )(degp, s20, s21, b2.reshape(1, d_out))

    return out


# speedup vs baseline: 7.7524x; 1.3175x over previous
import functools

import jax
import jax.numpy as jnp
from jax import lax
from jax.experimental import pallas as pl
from jax.experimental.pallas import tpu as pltpu
from jax.experimental.pallas import tpu_sc as plsc

N = 10000
E = 160000
NT = 16
NC = 2
INIT_TILES = 10
INIT_ROWS = N // INIT_TILES


def _sc_mesh():
    return plsc.VectorSubcoreMesh(core_axis_name="c", subcore_axis_name="s",
                                  num_cores=NC, num_subcores=NT)


_DEG_E_TILE = E // (NC * NT)
_DEG_STAGE = _DEG_E_TILE + 16
_DEG_VREGS = -(-_DEG_E_TILE // 16)


@functools.cache
def _make_deg_kernel():
    return functools.partial(
        pl.kernel,
        out_type=jax.ShapeDtypeStruct((NC * NT * N,), jnp.float32),
        mesh=_sc_mesh(),
        scratch_types=[
            pltpu.VMEM((_DEG_STAGE,), jnp.int32),
            pltpu.VMEM((N,), jnp.float32),
        ],
        compiler_params=pltpu.CompilerParams(needs_layout_passes=False),
    )(_deg_body)


def _deg_body(dstp_hbm, zeros_hbm, out_hbm, ebuf, dloc):
    c = lax.axis_index("c")
    s = lax.axis_index("s")
    wid = c * NT + s
    lane = lax.iota(jnp.int32, 16)

    pltpu.sync_copy(zeros_hbm, dloc)
    tile_base = pl.multiple_of(wid * _DEG_E_TILE, 8)
    pltpu.sync_copy(dstp_hbm.at[pl.ds(tile_base, _DEG_STAGE)], ebuf)

    def body(j, carry):
        idx = ebuf[pl.ds(j * 16, 16)]
        valid = lane < jnp.minimum(_DEG_E_TILE - j * 16, 16)
        cnt, last = plsc.scan_count(idx, mask=valid)
        plsc.addupdate_scatter(dloc, (idx,), cnt.astype(jnp.float32),
                               mask=last)
        return carry

    lax.fori_loop(0, _DEG_VREGS, body, 0)
    out_base = pl.multiple_of(wid * N, 8)
    pltpu.sync_copy(dloc, out_hbm.at[pl.ds(out_base, N)])


def _prop_tile_loop(g_hbm, src_hbm, dst_hbm, acc, bufs,
                    tile_base, num_batches, eb):
    (src_v0, src_v1, dst_v0, dst_v1, rows_v0, rows_v1, sem0, sem1) = bufs

    def start(i, src_v, dst_v, rows_v, sem):
        base = pl.multiple_of(tile_base + i * eb, 8)
        pltpu.sync_copy(src_hbm.at[pl.ds(base, eb)], src_v)
        pltpu.sync_copy(dst_hbm.at[pl.ds(base, eb)], dst_v)
        pltpu.async_copy(g_hbm.at[src_v], rows_v, sem)

    def finish(src_v, dst_v, rows_v, sem):
        pltpu.make_async_copy(g_hbm.at[src_v], rows_v, sem).wait()
        pltpu.sync_copy(rows_v, acc.at[dst_v], add=True)

    start(0, src_v0, dst_v0, rows_v0, sem0)

    def body(k, carry):
        start(2 * k + 1, src_v1, dst_v1, rows_v1, sem1)
        finish(src_v0, dst_v0, rows_v0, sem0)
        start(2 * k + 2, src_v0, dst_v0, rows_v0, sem0)
        finish(src_v1, dst_v1, rows_v1, sem1)
        return carry

    lax.fori_loop(0, (num_batches - 1) // 2, body, 0)
    finish(src_v0, dst_v0, rows_v0, sem0)


def _spmem_init(src_hbm_rows, acc, s):
    @pl.when(s < INIT_TILES)
    def _():
        row0 = s * INIT_ROWS
        pltpu.sync_copy(src_hbm_rows.at[pl.ds(row0, INIT_ROWS)],
                        acc.at[pl.ds(row0, INIT_ROWS)])


def _spmem_writeout(acc, out_hbm, s):
    @pl.when(s < INIT_TILES)
    def _():
        row0 = s * INIT_ROWS
        pltpu.sync_copy(acc.at[pl.ds(row0, INIT_ROWS)],
                        out_hbm.at[pl.ds(row0, INIT_ROWS)])


EB1 = 40
EB2 = 80


@functools.cache
def _make_prop_edge(dh):
    @functools.partial(
        pl.kernel,
        out_type=(
            jax.ShapeDtypeStruct((N, dh), jnp.float32),
            jax.ShapeDtypeStruct((N, dh), jnp.float32),
        ),
        mesh=_sc_mesh(),
        scratch_types=[
            pltpu.VMEM((EB1,), jnp.int32),
            pltpu.VMEM((EB1,), jnp.int32),
            pltpu.VMEM((EB1,), jnp.int32),
            pltpu.VMEM((EB1,), jnp.int32),
            pltpu.VMEM((EB1, dh), jnp.float32),
            pltpu.VMEM((EB1, dh), jnp.float32),
            pltpu.VMEM_SHARED((N, dh), jnp.float32),
            pltpu.SemaphoreType.DMA,
            pltpu.SemaphoreType.DMA,
        ],
        compiler_params=pltpu.CompilerParams(needs_layout_passes=False),
    )
    def prop(g_hbm, zeros_hbm, src_hbm, dst_hbm, out0, out1,
             src_v0, src_v1, dst_v0, dst_v1, rows_v0, rows_v1,
             acc, sem0, sem1):
        bufs = (src_v0, src_v1, dst_v0, dst_v1, rows_v0, rows_v1, sem0, sem1)
        c = lax.axis_index("c")
        s = lax.axis_index("s")
        e_per_tile = E // (NC * NT)

        def run(init_hbm, out_hbm, core_base):
            _spmem_init(init_hbm, acc, s)
            plsc.subcore_barrier()
            _prop_tile_loop(g_hbm, src_hbm, dst_hbm, acc, bufs,
                            core_base + s * e_per_tile,
                            e_per_tile // EB1, EB1)
            plsc.subcore_barrier()
            _spmem_writeout(acc, out_hbm, s)

        @pl.when(c == 0)
        def _():
            run(g_hbm, out0, 0)

        @pl.when(c == 1)
        def _():
            run(zeros_hbm, out1, E // 2)

    return prop


@functools.cache
def _make_prop_feat(dh):
    @functools.partial(
        pl.kernel,
        out_type=(
            jax.ShapeDtypeStruct((N, dh), jnp.float32),
            jax.ShapeDtypeStruct((N, dh), jnp.float32),
        ),
        mesh=_sc_mesh(),
        scratch_types=[
            pltpu.VMEM((EB2,), jnp.int32),
            pltpu.VMEM((EB2,), jnp.int32),
            pltpu.VMEM((EB2,), jnp.int32),
            pltpu.VMEM((EB2,), jnp.int32),
            pltpu.VMEM((EB2, dh), jnp.float32),
            pltpu.VMEM((EB2, dh), jnp.float32),
            pltpu.VMEM_SHARED((N, dh), jnp.float32),
            pltpu.SemaphoreType.DMA,
            pltpu.SemaphoreType.DMA,
        ],
        compiler_params=pltpu.CompilerParams(needs_layout_passes=False),
    )
    def prop(g0_hbm, g1_hbm, src_hbm, dst_hbm, out0, out1,
             src_v0, src_v1, dst_v0, dst_v1, rows_v0, rows_v1,
             acc, sem0, sem1):
        bufs = (src_v0, src_v1, dst_v0, dst_v1, rows_v0, rows_v1, sem0, sem1)
        c = lax.axis_index("c")
        s = lax.axis_index("s")
        e_per_tile = E // NT

        def run(g_hbm, out_hbm):
            _spmem_init(g_hbm, acc, s)
            plsc.subcore_barrier()
            _prop_tile_loop(g_hbm, src_hbm, dst_hbm, acc, bufs,
                            s * e_per_tile, e_per_tile // EB2, EB2)
            plsc.subcore_barrier()
            _spmem_writeout(acc, out_hbm, s)

        @pl.when(c == 0)
        def _():
            run(g0_hbm, out0)

        @pl.when(c == 1)
        def _():
            run(g1_hbm, out1)

    return prop


_RB = 1000


def _dinv(degp_ref):
    deg = 1.0 + jnp.sum(degp_ref[:, :, 0], axis=0)
    return lax.rsqrt(deg)


def _k1_body(degp_ref, x_ref, w1_ref, g_ref):
    dinv = _dinv(degp_ref)
    hw = jnp.dot(x_ref[:, :], w1_ref[:, :], preferred_element_type=jnp.float32)
    g_ref[:, :] = hw * dinv[:, None]


def _k2_body(degp_ref, sa_ref, sb_ref, b1_ref, w2_ref,
             g0_ref, g1_ref):
    dinv = _dinv(degp_ref)
    sfull = sa_ref[:, :] + sb_ref[:, :]
    h1 = jax.nn.relu(sfull * dinv[:, None] + b1_ref[0, :][None, :])
    g = jnp.dot(h1, w2_ref[:, :], preferred_element_type=jnp.float32)
    g = g * dinv[:, None]
    h = g.shape[1] // 2
    g0_ref[:, :] = g[:, :h]
    g1_ref[:, :] = g[:, h:]


def _k3_body(degp_ref, s0_ref, s1_ref, b2_ref, out_ref):
    dinv = _dinv(degp_ref)
    sfull = jnp.concatenate([s0_ref[:, :], s1_ref[:, :]], axis=1)
    out_ref[:, :] = sfull * dinv[:, None] + b2_ref[0, :][None, :]


def _row_spec(w):
    return pl.BlockSpec((_RB, w), lambda i: (i, 0))


def _degp_spec():
    return pl.BlockSpec((NC * NT, _RB, 1), lambda i: (0, i, 0))


def _full_spec(shape):
    return pl.BlockSpec(shape, lambda i: tuple(0 for _ in shape))


def kernel(x, edge_index, cache_name, W1, b1, W2, b2):
    del cache_name
    n, d_in = x.shape
    d_hid = W1.shape[1]
    d_out = W2.shape[1]
    assert n == N and edge_index.shape == (2, E)
    grid = (n // _RB,)

    src = edge_index[0].astype(jnp.int32)
    dst = edge_index[1].astype(jnp.int32)
    dst_pad = jnp.concatenate([dst, jnp.zeros((16,), jnp.int32)])
    zeros_nh = jnp.zeros((N, d_hid), jnp.float32)

    zeros_flat = jnp.zeros((N,), jnp.float32)
    degp = _make_deg_kernel()(dst_pad, zeros_flat)
    degp = degp.reshape(NC * NT, N, 1)

    g1 = pl.pallas_call(
        _k1_body,
        grid=grid,
        in_specs=[_degp_spec(), _row_spec(d_in), _full_spec(W1.shape)],
        out_specs=_row_spec(d_hid),
        out_shape=jax.ShapeDtypeStruct((n, d_hid), jnp.float32),
    )(degp, x, W1)

    s1a, s1b = _make_prop_edge(d_hid)(g1, zeros_nh, src, dst)

    g20, g21 = pl.pallas_call(
        _k2_body,
        grid=grid,
        in_specs=[_degp_spec(), _row_spec(d_hid), _row_spec(d_hid),
                  _full_spec((1, d_hid)), _full_spec(W2.shape)],
        out_specs=[_row_spec(d_out // 2), _row_spec(d_out // 2)],
        out_shape=[jax.ShapeDtypeStruct((n, d_out // 2), jnp.float32)] * 2,
    )(degp, s1a, s1b, b1.reshape(1, d_hid), W2)

    s20, s21 = _make_prop_feat(d_out // 2)(g20, g21, src, dst)

    out = pl.pallas_call(
        _k3_body,
        grid=grid,
        in_specs=[_degp_spec(), _row_spec(d_out // 2), _row_spec(d_out // 2),
                  _full_spec((1, d_out))],
        out_specs=_row_spec(d_out),
        out_shape=jax.ShapeDtypeStruct((n, d_out), jnp.float32),
    )(degp, s20, s21, b2.reshape(1, d_out))

    return out


# speedup vs baseline: 13.4855x; 1.7395x over previous
import functools

import jax
import jax.numpy as jnp
from jax import lax
from jax.experimental import pallas as pl
from jax.experimental.pallas import tpu as pltpu
from jax.experimental.pallas import tpu_sc as plsc

N = 10000
E = 160000
NT = 16
NC = 2
INIT_TILES = 10
INIT_ROWS = N // INIT_TILES


def _sc_mesh():
    return plsc.VectorSubcoreMesh(core_axis_name="c", subcore_axis_name="s",
                                  num_cores=NC, num_subcores=NT)


_DEG_E_TILE = E // (NC * NT)
_DEG_STAGE = _DEG_E_TILE + 16
_DEG_VREGS = -(-_DEG_E_TILE // 16)


_DEG_ROWS = 80


@functools.cache
def _make_deg_kernel():
    return functools.partial(
        pl.kernel,
        out_type=jax.ShapeDtypeStruct((NC * NT, _DEG_ROWS, 128), jnp.float32),
        mesh=_sc_mesh(),
        scratch_types=[
            pltpu.VMEM((_DEG_STAGE,), jnp.int32),
            pltpu.VMEM((_DEG_ROWS, 128), jnp.float32),
        ],
        compiler_params=pltpu.CompilerParams(needs_layout_passes=False),
    )(_deg_body)


def _deg_body(dstp_hbm, zeros_hbm, out_hbm, ebuf, dloc):
    c = lax.axis_index("c")
    s = lax.axis_index("s")
    wid = c * NT + s
    lane = lax.iota(jnp.int32, 16)

    pltpu.sync_copy(zeros_hbm.at[pl.ds(0, _DEG_ROWS)], dloc)
    tile_base = pl.multiple_of(wid * _DEG_E_TILE, 8)
    pltpu.sync_copy(dstp_hbm.at[pl.ds(tile_base, _DEG_STAGE)], ebuf)

    def body(j, carry):
        idx = ebuf[pl.ds(j * 16, 16)]
        valid = lane < jnp.minimum(_DEG_E_TILE - j * 16, 16)
        cnt, last = plsc.scan_count(idx, mask=valid)
        plsc.addupdate_scatter(
            dloc,
            (lax.shift_right_logical(idx, 7), lax.bitwise_and(idx, 127)),
            cnt.astype(jnp.float32), mask=last)
        return carry

    lax.fori_loop(0, _DEG_VREGS, body, 0)
    pltpu.sync_copy(dloc, out_hbm.at[wid])


def _prop_tile_loop(g_hbm, src_hbm, dst_hbm, acc, bufs,
                    tile_base, num_batches, eb):
    (src_v0, src_v1, dst_v0, dst_v1, rows_v0, rows_v1, sem0, sem1) = bufs

    def start(i, src_v, dst_v, rows_v, sem):
        base = pl.multiple_of(tile_base + i * eb, 8)
        pltpu.sync_copy(src_hbm.at[pl.ds(base, eb)], src_v)
        pltpu.sync_copy(dst_hbm.at[pl.ds(base, eb)], dst_v)
        pltpu.async_copy(g_hbm.at[src_v], rows_v, sem)

    def finish(src_v, dst_v, rows_v, sem):
        pltpu.make_async_copy(g_hbm.at[src_v], rows_v, sem).wait()
        pltpu.sync_copy(rows_v, acc.at[dst_v], add=True)

    start(0, src_v0, dst_v0, rows_v0, sem0)

    def body(k, carry):
        start(2 * k + 1, src_v1, dst_v1, rows_v1, sem1)
        finish(src_v0, dst_v0, rows_v0, sem0)
        start(2 * k + 2, src_v0, dst_v0, rows_v0, sem0)
        finish(src_v1, dst_v1, rows_v1, sem1)
        return carry

    lax.fori_loop(0, (num_batches - 1) // 2, body, 0)
    finish(src_v0, dst_v0, rows_v0, sem0)


def _spmem_init(src_hbm_rows, acc, s):
    @pl.when(s < INIT_TILES)
    def _():
        row0 = s * INIT_ROWS
        pltpu.sync_copy(src_hbm_rows.at[pl.ds(row0, INIT_ROWS)],
                        acc.at[pl.ds(row0, INIT_ROWS)])


def _spmem_writeout(acc, out_hbm, s):
    @pl.when(s < INIT_TILES)
    def _():
        row0 = s * INIT_ROWS
        pltpu.sync_copy(acc.at[pl.ds(row0, INIT_ROWS)],
                        out_hbm.at[pl.ds(row0, INIT_ROWS)])


EB1 = 40
EB2 = 80


@functools.cache
def _make_prop_edge(dh):
    @functools.partial(
        pl.kernel,
        out_type=(
            jax.ShapeDtypeStruct((N, dh), jnp.float32),
            jax.ShapeDtypeStruct((N, dh), jnp.float32),
        ),
        mesh=_sc_mesh(),
        scratch_types=[
            pltpu.VMEM((EB1,), jnp.int32),
            pltpu.VMEM((EB1,), jnp.int32),
            pltpu.VMEM((EB1,), jnp.int32),
            pltpu.VMEM((EB1,), jnp.int32),
            pltpu.VMEM((EB1, dh), jnp.float32),
            pltpu.VMEM((EB1, dh), jnp.float32),
            pltpu.VMEM_SHARED((N, dh), jnp.float32),
            pltpu.SemaphoreType.DMA,
            pltpu.SemaphoreType.DMA,
        ],
        compiler_params=pltpu.CompilerParams(needs_layout_passes=False),
    )
    def prop(g_hbm, zeros_hbm, src_hbm, dst_hbm, out0, out1,
             src_v0, src_v1, dst_v0, dst_v1, rows_v0, rows_v1,
             acc, sem0, sem1):
        bufs = (src_v0, src_v1, dst_v0, dst_v1, rows_v0, rows_v1, sem0, sem1)
        c = lax.axis_index("c")
        s = lax.axis_index("s")
        e_per_tile = E // (NC * NT)

        def run(init_hbm, out_hbm, core_base):
            _spmem_init(init_hbm, acc, s)
            plsc.subcore_barrier()
            _prop_tile_loop(g_hbm, src_hbm, dst_hbm, acc, bufs,
                            core_base + s * e_per_tile,
                            e_per_tile // EB1, EB1)
            plsc.subcore_barrier()
            _spmem_writeout(acc, out_hbm, s)

        @pl.when(c == 0)
        def _():
            run(g_hbm, out0, 0)

        @pl.when(c == 1)
        def _():
            run(zeros_hbm, out1, E // 2)

    return prop


@functools.cache
def _make_prop_feat(dh):
    @functools.partial(
        pl.kernel,
        out_type=(
            jax.ShapeDtypeStruct((N, dh), jnp.float32),
            jax.ShapeDtypeStruct((N, dh), jnp.float32),
        ),
        mesh=_sc_mesh(),
        scratch_types=[
            pltpu.VMEM((EB2,), jnp.int32),
            pltpu.VMEM((EB2,), jnp.int32),
            pltpu.VMEM((EB2,), jnp.int32),
            pltpu.VMEM((EB2,), jnp.int32),
            pltpu.VMEM((EB2, dh), jnp.float32),
            pltpu.VMEM((EB2, dh), jnp.float32),
            pltpu.VMEM_SHARED((N, dh), jnp.float32),
            pltpu.SemaphoreType.DMA,
            pltpu.SemaphoreType.DMA,
        ],
        compiler_params=pltpu.CompilerParams(needs_layout_passes=False),
    )
    def prop(g0_hbm, g1_hbm, src_hbm, dst_hbm, out0, out1,
             src_v0, src_v1, dst_v0, dst_v1, rows_v0, rows_v1,
             acc, sem0, sem1):
        bufs = (src_v0, src_v1, dst_v0, dst_v1, rows_v0, rows_v1, sem0, sem1)
        c = lax.axis_index("c")
        s = lax.axis_index("s")
        e_per_tile = E // NT

        def run(g_hbm, out_hbm):
            _spmem_init(g_hbm, acc, s)
            plsc.subcore_barrier()
            _prop_tile_loop(g_hbm, src_hbm, dst_hbm, acc, bufs,
                            s * e_per_tile, e_per_tile // EB2, EB2)
            plsc.subcore_barrier()
            _spmem_writeout(acc, out_hbm, s)

        @pl.when(c == 0)
        def _():
            run(g0_hbm, out0)

        @pl.when(c == 1)
        def _():
            run(g1_hbm, out1)

    return prop


_RB = 1024


def _k1_body(degp_ref, x_ref, w1_ref, g_ref, dinvb_ref):
    deg2d = 1.0 + jnp.sum(degp_ref[:, :, :], axis=0)
    dv = lax.rsqrt(deg2d)
    dv3 = jnp.broadcast_to(dv[:, None, :], (8, 128, 128)).reshape(_RB, 128)
    rowm = lax.broadcasted_iota(jnp.int32, (_RB, 128), 0)
    lanem = lax.broadcasted_iota(jnp.int32, (_RB, 128), 1)
    sel = (lanem == (rowm & 127)).astype(jnp.float32)
    dinv_col = jnp.sum(dv3 * sel, axis=1, keepdims=True)
    hw = jnp.dot(x_ref[:, :], w1_ref[:, :], preferred_element_type=jnp.float32)
    g_ref[:, :] = hw * dinv_col
    dinvb_ref[:, :] = jnp.broadcast_to(dinv_col, (_RB, hw.shape[1]))


def _k2_body(dinvb_ref, sa_ref, sb_ref, b1_ref, w2_ref,
             g0_ref, g1_ref):
    dinvb = dinvb_ref[:, :]
    sfull = sa_ref[:, :] + sb_ref[:, :]
    h1 = jax.nn.relu(sfull * dinvb + b1_ref[0, :][None, :])
    g = jnp.dot(h1, w2_ref[:, :], preferred_element_type=jnp.float32)
    h = g.shape[1] // 2
    g0_ref[:, :] = g[:, :h] * dinvb
    g1_ref[:, :] = g[:, h:] * dinvb


def _k3_body(dinvb_ref, s0_ref, s1_ref, b2_ref, out_ref):
    dinvb = dinvb_ref[:, :]
    sfull = jnp.concatenate([s0_ref[:, :] * dinvb, s1_ref[:, :] * dinvb],
                            axis=1)
    out_ref[:, :] = sfull + b2_ref[0, :][None, :]


def _row_spec(w):
    return pl.BlockSpec((_RB, w), lambda i: (i, 0))


def _degp_spec():
    return pl.BlockSpec((NC * NT, _RB // 128, 128), lambda i: (0, i, 0))


def _full_spec(shape):
    return pl.BlockSpec(shape, lambda i: tuple(0 for _ in shape))


def kernel(x, edge_index, cache_name, W1, b1, W2, b2):
    del cache_name
    n, d_in = x.shape
    d_hid = W1.shape[1]
    d_out = W2.shape[1]
    assert n == N and edge_index.shape == (2, E)
    grid = (-(-n // _RB),)

    src = edge_index[0].astype(jnp.int32)
    dst = edge_index[1].astype(jnp.int32)
    dst_pad = jnp.concatenate([dst, jnp.zeros((16,), jnp.int32)])
    zeros_nh = jnp.zeros((N, d_hid), jnp.float32)

    degp = _make_deg_kernel()(dst_pad, zeros_nh)

    g1, dinvb = pl.pallas_call(
        _k1_body,
        grid=grid,
        in_specs=[_degp_spec(), _row_spec(d_in), _full_spec(W1.shape)],
        out_specs=[_row_spec(d_hid), _row_spec(d_hid)],
        out_shape=[jax.ShapeDtypeStruct((n, d_hid), jnp.float32)] * 2,
    )(degp, x, W1)

    s1a, s1b = _make_prop_edge(d_hid)(g1, zeros_nh, src, dst)

    g20, g21 = pl.pallas_call(
        _k2_body,
        grid=grid,
        in_specs=[_row_spec(d_hid), _row_spec(d_hid), _row_spec(d_hid),
                  _full_spec((1, d_hid)), _full_spec(W2.shape)],
        out_specs=[_row_spec(d_out // 2), _row_spec(d_out // 2)],
        out_shape=[jax.ShapeDtypeStruct((n, d_out // 2), jnp.float32)] * 2,
    )(dinvb, s1a, s1b, b1.reshape(1, d_hid), W2)

    s20, s21 = _make_prop_feat(d_out // 2)(g20, g21, src, dst)

    out = pl.pallas_call(
        _k3_body,
        grid=grid,
        in_specs=[_row_spec(d_out // 2), _row_spec(d_out // 2),
                  _row_spec(d_out // 2), _full_spec((1, d_out))],
        out_specs=_row_spec(d_out),
        out_shape=jax.ShapeDtypeStruct((n, d_out), jnp.float32),
    )(dinvb, s20, s21, b2.reshape(1, d_out))

    return out


# speedup vs baseline: 17.8781x; 1.3257x over previous
import functools

import jax
import jax.numpy as jnp
from jax import lax
from jax.experimental import pallas as pl
from jax.experimental.pallas import tpu as pltpu
from jax.experimental.pallas import tpu_sc as plsc

N = 10000
E = 160000
NT = 16
NC = 2
INIT_TILES = 10
INIT_ROWS = N // INIT_TILES


def _sc_mesh():
    return plsc.VectorSubcoreMesh(core_axis_name="c", subcore_axis_name="s",
                                  num_cores=NC, num_subcores=NT)


_DEG_E_TILE = E // (NC * NT)
_DEG_STAGE = _DEG_E_TILE + 16
_DEG_VREGS = -(-_DEG_E_TILE // 16)


_DEG_ROWS = 80


@functools.cache
def _make_deg_kernel():
    return functools.partial(
        pl.kernel,
        out_type=jax.ShapeDtypeStruct((NC * NT, _DEG_ROWS, 128), jnp.float32),
        mesh=_sc_mesh(),
        scratch_types=[
            pltpu.VMEM((_DEG_STAGE,), jnp.int32),
            pltpu.VMEM((_DEG_ROWS, 128), jnp.float32),
        ],
        compiler_params=pltpu.CompilerParams(needs_layout_passes=False),
    )(_deg_body)


def _deg_body(dstp_hbm, zeros_hbm, out_hbm, ebuf, dloc):
    c = lax.axis_index("c")
    s = lax.axis_index("s")
    wid = c * NT + s
    lane = lax.iota(jnp.int32, 16)

    pltpu.sync_copy(zeros_hbm.at[pl.ds(0, _DEG_ROWS)], dloc)
    tile_base = pl.multiple_of(wid * _DEG_E_TILE, 8)
    pltpu.sync_copy(dstp_hbm.at[pl.ds(tile_base, _DEG_STAGE)], ebuf)

    def body(j, carry):
        idx = ebuf[pl.ds(j * 16, 16)]
        valid = lane < jnp.minimum(_DEG_E_TILE - j * 16, 16)
        cnt, last = plsc.scan_count(idx, mask=valid)
        plsc.addupdate_scatter(
            dloc,
            (lax.shift_right_logical(idx, 7), lax.bitwise_and(idx, 127)),
            cnt.astype(jnp.float32), mask=last)
        return carry

    lax.fori_loop(0, _DEG_VREGS, body, 0)
    pltpu.sync_copy(dloc, out_hbm.at[wid])


def _prop_tile_loop(g_hbm, src_hbm, dst_hbm, acc, bufs, tbufs,
                    tile_base, nb, eb, teb):
    (src_v0, src_v1, dst_v0, dst_v1, rows_v0, rows_v1, sem0, sem1) = bufs

    def start(i, src_v, dst_v, rows_v, sem):
        base = pl.multiple_of(tile_base + i * eb, 8)
        pltpu.sync_copy(src_hbm.at[pl.ds(base, eb)], src_v)
        pltpu.sync_copy(dst_hbm.at[pl.ds(base, eb)], dst_v)
        pltpu.async_copy(g_hbm.at[src_v], rows_v, sem)

    def finish(src_v, dst_v, rows_v, sem):
        pltpu.make_async_copy(g_hbm.at[src_v], rows_v, sem).wait()
        pltpu.sync_copy(rows_v, acc.at[dst_v], add=True)

    nbp = nb if nb % 2 == 1 else nb - 1
    start(0, src_v0, dst_v0, rows_v0, sem0)

    def body(k, carry):
        start(2 * k + 1, src_v1, dst_v1, rows_v1, sem1)
        finish(src_v0, dst_v0, rows_v0, sem0)
        start(2 * k + 2, src_v0, dst_v0, rows_v0, sem0)
        finish(src_v1, dst_v1, rows_v1, sem1)
        return carry

    lax.fori_loop(0, (nbp - 1) // 2, body, 0)
    finish(src_v0, dst_v0, rows_v0, sem0)
    if nb != nbp:
        start(nb - 1, src_v0, dst_v0, rows_v0, sem0)
        finish(src_v0, dst_v0, rows_v0, sem0)
    if teb:
        tsrc_v, tdst_v, trows_v = tbufs
        base = pl.multiple_of(tile_base + nb * eb, 8)
        pltpu.sync_copy(src_hbm.at[pl.ds(base, teb)], tsrc_v)
        pltpu.sync_copy(dst_hbm.at[pl.ds(base, teb)], tdst_v)
        pltpu.async_copy(g_hbm.at[tsrc_v], trows_v, sem0).wait()
        pltpu.sync_copy(trows_v, acc.at[tdst_v], add=True)


def _spmem_init(src_hbm_rows, acc, s):
    @pl.when(s < INIT_TILES)
    def _():
        row0 = s * INIT_ROWS
        pltpu.sync_copy(src_hbm_rows.at[pl.ds(row0, INIT_ROWS)],
                        acc.at[pl.ds(row0, INIT_ROWS)])


def _spmem_writeout(acc, out_hbm, s):
    @pl.when(s < INIT_TILES)
    def _():
        row0 = s * INIT_ROWS
        pltpu.sync_copy(acc.at[pl.ds(row0, INIT_ROWS)],
                        out_hbm.at[pl.ds(row0, INIT_ROWS)])


EB = 128


def _prop_scratch(dh, teb):
    return [
        pltpu.VMEM((EB,), jnp.int32),
        pltpu.VMEM((EB,), jnp.int32),
        pltpu.VMEM((EB,), jnp.int32),
        pltpu.VMEM((EB,), jnp.int32),
        pltpu.VMEM((EB, dh), jnp.float32),
        pltpu.VMEM((EB, dh), jnp.float32),
        pltpu.VMEM((teb,), jnp.int32),
        pltpu.VMEM((teb,), jnp.int32),
        pltpu.VMEM((teb, dh), jnp.float32),
        pltpu.VMEM_SHARED((N, dh), jnp.float32),
        pltpu.SemaphoreType.DMA,
        pltpu.SemaphoreType.DMA,
    ]


@functools.cache
def _make_prop_edge(dh):
    e_tile = E // (NC * NT)
    nb, teb = e_tile // EB, e_tile % EB

    @functools.partial(
        pl.kernel,
        out_type=(
            jax.ShapeDtypeStruct((N, dh), jnp.float32),
            jax.ShapeDtypeStruct((N, dh), jnp.float32),
        ),
        mesh=_sc_mesh(),
        scratch_types=_prop_scratch(dh, teb),
        compiler_params=pltpu.CompilerParams(needs_layout_passes=False),
    )
    def prop(g_hbm, zeros_hbm, src_hbm, dst_hbm, out0, out1,
             src_v0, src_v1, dst_v0, dst_v1, rows_v0, rows_v1,
             tsrc_v, tdst_v, trows_v, acc, sem0, sem1):
        bufs = (src_v0, src_v1, dst_v0, dst_v1, rows_v0, rows_v1, sem0, sem1)
        tbufs = (tsrc_v, tdst_v, trows_v)
        c = lax.axis_index("c")
        s = lax.axis_index("s")

        def run(init_hbm, out_hbm, core_base):
            _spmem_init(init_hbm, acc, s)
            plsc.subcore_barrier()
            _prop_tile_loop(g_hbm, src_hbm, dst_hbm, acc, bufs, tbufs,
                            core_base + s * e_tile, nb, EB, teb)
            plsc.subcore_barrier()
            _spmem_writeout(acc, out_hbm, s)

        @pl.when(c == 0)
        def _():
            run(g_hbm, out0, 0)

        @pl.when(c == 1)
        def _():
            run(zeros_hbm, out1, E // 2)

    return prop


@functools.cache
def _make_prop_feat(dh):
    e_tile = E // NT
    nb, teb = e_tile // EB, e_tile % EB

    @functools.partial(
        pl.kernel,
        out_type=(
            jax.ShapeDtypeStruct((N, dh), jnp.float32),
            jax.ShapeDtypeStruct((N, dh), jnp.float32),
        ),
        mesh=_sc_mesh(),
        scratch_types=_prop_scratch(dh, teb),
        compiler_params=pltpu.CompilerParams(needs_layout_passes=False),
    )
    def prop(g0_hbm, g1_hbm, src_hbm, dst_hbm, out0, out1,
             src_v0, src_v1, dst_v0, dst_v1, rows_v0, rows_v1,
             tsrc_v, tdst_v, trows_v, acc, sem0, sem1):
        bufs = (src_v0, src_v1, dst_v0, dst_v1, rows_v0, rows_v1, sem0, sem1)
        tbufs = (tsrc_v, tdst_v, trows_v)
        c = lax.axis_index("c")
        s = lax.axis_index("s")

        def run(g_hbm, out_hbm):
            _spmem_init(g_hbm, acc, s)
            plsc.subcore_barrier()
            _prop_tile_loop(g_hbm, src_hbm, dst_hbm, acc, bufs, tbufs,
                            s * e_tile, nb, EB, teb)
            plsc.subcore_barrier()
            _spmem_writeout(acc, out_hbm, s)

        @pl.when(c == 0)
        def _():
            run(g0_hbm, out0)

        @pl.when(c == 1)
        def _():
            run(g1_hbm, out1)

    return prop


_RB = 1024


def _k1_body(degp_ref, x_ref, w1_ref, g_ref, dinvb_ref):
    deg2d = 1.0 + jnp.sum(degp_ref[:, :, :], axis=0)
    dv = lax.rsqrt(deg2d)
    dv3 = jnp.broadcast_to(dv[:, None, :], (8, 128, 128)).reshape(_RB, 128)
    rowm = lax.broadcasted_iota(jnp.int32, (_RB, 128), 0)
    lanem = lax.broadcasted_iota(jnp.int32, (_RB, 128), 1)
    sel = (lanem == (rowm & 127)).astype(jnp.float32)
    dinv_col = jnp.sum(dv3 * sel, axis=1, keepdims=True)
    hw = jnp.dot(x_ref[:, :], w1_ref[:, :], preferred_element_type=jnp.float32)
    g_ref[:, :] = hw * dinv_col
    dinvb_ref[:, :] = jnp.broadcast_to(dinv_col, (_RB, hw.shape[1]))


def _k2_body(dinvb_ref, sa_ref, sb_ref, b1_ref, w2_ref,
             g0_ref, g1_ref):
    dinvb = dinvb_ref[:, :]
    sfull = sa_ref[:, :] + sb_ref[:, :]
    h1 = jax.nn.relu(sfull * dinvb + b1_ref[0, :][None, :])
    g = jnp.dot(h1, w2_ref[:, :], preferred_element_type=jnp.float32)
    h = g.shape[1] // 2
    g0_ref[:, :] = g[:, :h] * dinvb
    g1_ref[:, :] = g[:, h:] * dinvb


def _k3_body(dinvb_ref, s0_ref, s1_ref, b2_ref, out_ref):
    dinvb = dinvb_ref[:, :]
    sfull = jnp.concatenate([s0_ref[:, :] * dinvb, s1_ref[:, :] * dinvb],
                            axis=1)
    out_ref[:, :] = sfull + b2_ref[0, :][None, :]


def _row_spec(w):
    return pl.BlockSpec((_RB, w), lambda i: (i, 0))


def _degp_spec():
    return pl.BlockSpec((NC * NT, _RB // 128, 128), lambda i: (0, i, 0))


def _full_spec(shape):
    return pl.BlockSpec(shape, lambda i: tuple(0 for _ in shape))


def kernel(x, edge_index, cache_name, W1, b1, W2, b2):
    del cache_name
    n, d_in = x.shape
    d_hid = W1.shape[1]
    d_out = W2.shape[1]
    assert n == N and edge_index.shape == (2, E)
    grid = (-(-n // _RB),)

    src = edge_index[0].astype(jnp.int32)
    dst = edge_index[1].astype(jnp.int32)
    dst_pad = jnp.concatenate([dst, jnp.zeros((16,), jnp.int32)])
    zeros_nh = jnp.zeros((N, d_hid), jnp.float32)

    degp = _make_deg_kernel()(dst_pad, zeros_nh)

    g1, dinvb = pl.pallas_call(
        _k1_body,
        grid=grid,
        in_specs=[_degp_spec(), _row_spec(d_in), _full_spec(W1.shape)],
        out_specs=[_row_spec(d_hid), _row_spec(d_hid)],
        out_shape=[jax.ShapeDtypeStruct((n, d_hid), jnp.float32)] * 2,
    )(degp, x, W1)

    s1a, s1b = _make_prop_edge(d_hid)(g1, zeros_nh, src, dst)

    g20, g21 = pl.pallas_call(
        _k2_body,
        grid=grid,
        in_specs=[_row_spec(d_hid), _row_spec(d_hid), _row_spec(d_hid),
                  _full_spec((1, d_hid)), _full_spec(W2.shape)],
        out_specs=[_row_spec(d_out // 2), _row_spec(d_out // 2)],
        out_shape=[jax.ShapeDtypeStruct((n, d_out // 2), jnp.float32)] * 2,
    )(dinvb, s1a, s1b, b1.reshape(1, d_hid), W2)

    s20, s21 = _make_prop_feat(d_out // 2)(g20, g21, src, dst)

    out = pl.pallas_call(
        _k3_body,
        grid=grid,
        in_specs=[_row_spec(d_out // 2), _row_spec(d_out // 2),
                  _row_spec(d_out // 2), _full_spec((1, d_out))],
        out_specs=_row_spec(d_out),
        out_shape=jax.ShapeDtypeStruct((n, d_out), jnp.float32),
    )(dinvb, s20, s21, b2.reshape(1, d_out))

    return out


# speedup vs baseline: 22.4686x; 1.2568x over previous
import functools

import jax
import jax.numpy as jnp
from jax import lax
from jax.experimental import pallas as pl
from jax.experimental.pallas import tpu as pltpu
from jax.experimental.pallas import tpu_sc as plsc

N = 10000
E = 160000
NT = 16
NC = 2
INIT_TILES = 10
INIT_ROWS = N // INIT_TILES


def _sc_mesh():
    return plsc.VectorSubcoreMesh(core_axis_name="c", subcore_axis_name="s",
                                  num_cores=NC, num_subcores=NT)


_DEG_E_TILE = E // (NC * NT)
_DEG_STAGE = _DEG_E_TILE + 16
_DEG_VREGS = -(-_DEG_E_TILE // 16)


_DEG_ROWS = 80


@functools.cache
def _make_deg_kernel():
    return functools.partial(
        pl.kernel,
        out_type=jax.ShapeDtypeStruct((NC * NT, _DEG_ROWS, 128), jnp.float32),
        mesh=_sc_mesh(),
        scratch_types=[
            pltpu.VMEM((_DEG_STAGE,), jnp.int32),
            pltpu.VMEM((_DEG_ROWS, 128), jnp.float32),
        ],
        compiler_params=pltpu.CompilerParams(needs_layout_passes=False),
    )(_deg_body)


def _deg_body(dstp_hbm, zeros_hbm, out_hbm, ebuf, dloc):
    c = lax.axis_index("c")
    s = lax.axis_index("s")
    wid = c * NT + s
    lane = lax.iota(jnp.int32, 16)

    pltpu.sync_copy(zeros_hbm.at[pl.ds(0, _DEG_ROWS)], dloc)
    tile_base = pl.multiple_of(wid * _DEG_E_TILE, 8)
    pltpu.sync_copy(dstp_hbm.at[pl.ds(tile_base, _DEG_STAGE)], ebuf)

    def body(j, carry):
        idx = ebuf[pl.ds(j * 16, 16)]
        valid = lane < jnp.minimum(_DEG_E_TILE - j * 16, 16)
        cnt, last = plsc.scan_count(idx, mask=valid)
        plsc.addupdate_scatter(
            dloc,
            (lax.shift_right_logical(idx, 7), lax.bitwise_and(idx, 127)),
            cnt.astype(jnp.float32), mask=last)
        return carry

    lax.fori_loop(0, _DEG_VREGS, body, 0)
    pltpu.sync_copy(dloc, out_hbm.at[wid])


def _prop_tile_loop(g_hbm, src_hbm, dst_hbm, acc, idxb, rowsb, tbufs,
                    tile_base, nb, eb, teb):
    rows_v0, rows_v1, semg0, semg1 = rowsb
    rows = (rows_v0, rows_v1)
    semg = (semg0, semg1)

    def issue_idx(i, j):
        src_v, dst_v, semi = idxb[j]
        base = pl.multiple_of(tile_base + i * eb, 8)
        pltpu.async_copy(src_hbm.at[pl.ds(base, eb)], src_v, semi)
        pltpu.async_copy(dst_hbm.at[pl.ds(base, eb)], dst_v, semi)

    def launch(i, j, r):
        src_v, dst_v, semi = idxb[j]
        base = pl.multiple_of(tile_base + i * eb, 8)
        pltpu.make_async_copy(src_hbm.at[pl.ds(base, eb)], src_v, semi).wait()
        pltpu.make_async_copy(dst_hbm.at[pl.ds(base, eb)], dst_v, semi).wait()
        pltpu.async_copy(g_hbm.at[src_v], rows[r], semg[r])

    def finish(i, j, r):
        src_v, dst_v, semi = idxb[j]
        pltpu.make_async_copy(g_hbm.at[src_v], rows[r], semg[r]).wait()
        pltpu.sync_copy(rows[r], acc.at[dst_v], add=True)

    def guarded(cond, fn, i, *a):
        @pl.when(cond)
        def _():
            fn(i, *a)

    for j in range(4):
        if j < nb:
            issue_idx(j, j)
    for j in range(2):
        if j < nb:
            launch(j, j, j % 2)

    def body(k, carry):
        for j in range(4):
            i = 4 * k + j
            guarded(i < nb, finish, i, j, j % 2)
            guarded(i + 4 < nb, issue_idx, i + 4, j)
            guarded(i + 2 < nb, launch, i + 2, (j + 2) % 4, j % 2)
        return carry

    lax.fori_loop(0, -(-nb // 4), body, 0)
    if teb:
        tsrc_v, tdst_v, trows_v = tbufs
        base = pl.multiple_of(tile_base + nb * eb, 8)
        pltpu.sync_copy(src_hbm.at[pl.ds(base, teb)], tsrc_v)
        pltpu.sync_copy(dst_hbm.at[pl.ds(base, teb)], tdst_v)
        pltpu.async_copy(g_hbm.at[tsrc_v], trows_v, semg0).wait()
        pltpu.sync_copy(trows_v, acc.at[tdst_v], add=True)


def _spmem_init(src_hbm_rows, acc, s):
    @pl.when(s < INIT_TILES)
    def _():
        row0 = s * INIT_ROWS
        pltpu.sync_copy(src_hbm_rows.at[pl.ds(row0, INIT_ROWS)],
                        acc.at[pl.ds(row0, INIT_ROWS)])


def _spmem_writeout(acc, out_hbm, s):
    @pl.when(s < INIT_TILES)
    def _():
        row0 = s * INIT_ROWS
        pltpu.sync_copy(acc.at[pl.ds(row0, INIT_ROWS)],
                        out_hbm.at[pl.ds(row0, INIT_ROWS)])


EB = 128


def _prop_scratch(dh, teb):
    ring = []
    for _ in range(4):
        ring += [
            pltpu.VMEM((EB,), jnp.int32),
            pltpu.VMEM((EB,), jnp.int32),
            pltpu.SemaphoreType.DMA,
        ]
    return ring + [
        pltpu.VMEM((EB, dh), jnp.float32),
        pltpu.VMEM((EB, dh), jnp.float32),
        pltpu.SemaphoreType.DMA,
        pltpu.SemaphoreType.DMA,
        pltpu.VMEM((teb,), jnp.int32),
        pltpu.VMEM((teb,), jnp.int32),
        pltpu.VMEM((teb, dh), jnp.float32),
        pltpu.VMEM_SHARED((N, dh), jnp.float32),
    ]


@functools.cache
def _make_prop_edge(dh):
    e_tile = E // (NC * NT)
    nb, teb = e_tile // EB, e_tile % EB

    @functools.partial(
        pl.kernel,
        out_type=(
            jax.ShapeDtypeStruct((N, dh), jnp.float32),
            jax.ShapeDtypeStruct((N, dh), jnp.float32),
        ),
        mesh=_sc_mesh(),
        scratch_types=_prop_scratch(dh, teb),
        compiler_params=pltpu.CompilerParams(needs_layout_passes=False),
    )
    def prop(g_hbm, zeros_hbm, src_hbm, dst_hbm, out0, out1,
             *scr):
        idxb = [tuple(scr[3 * j:3 * j + 3]) for j in range(4)]
        rowsb = scr[12:16]
        tbufs = scr[16:19]
        acc = scr[19]
        c = lax.axis_index("c")
        s = lax.axis_index("s")

        def run(init_hbm, out_hbm, core_base):
            _spmem_init(init_hbm, acc, s)
            plsc.subcore_barrier()
            _prop_tile_loop(g_hbm, src_hbm, dst_hbm, acc, idxb, rowsb,
                            tbufs, core_base + s * e_tile, nb, EB, teb)
            plsc.subcore_barrier()
            _spmem_writeout(acc, out_hbm, s)

        @pl.when(c == 0)
        def _():
            run(g_hbm, out0, 0)

        @pl.when(c == 1)
        def _():
            run(zeros_hbm, out1, E // 2)

    return prop


@functools.cache
def _make_prop_feat(dh):
    e_tile = E // NT
    nb, teb = e_tile // EB, e_tile % EB

    @functools.partial(
        pl.kernel,
        out_type=(
            jax.ShapeDtypeStruct((N, dh), jnp.float32),
            jax.ShapeDtypeStruct((N, dh), jnp.float32),
        ),
        mesh=_sc_mesh(),
        scratch_types=_prop_scratch(dh, teb),
        compiler_params=pltpu.CompilerParams(needs_layout_passes=False),
    )
    def prop(g0_hbm, g1_hbm, src_hbm, dst_hbm, out0, out1,
             *scr):
        idxb = [tuple(scr[3 * j:3 * j + 3]) for j in range(4)]
        rowsb = scr[12:16]
        tbufs = scr[16:19]
        acc = scr[19]
        c = lax.axis_index("c")
        s = lax.axis_index("s")

        def run(g_hbm, out_hbm):
            _spmem_init(g_hbm, acc, s)
            plsc.subcore_barrier()
            _prop_tile_loop(g_hbm, src_hbm, dst_hbm, acc, idxb, rowsb,
                            tbufs, s * e_tile, nb, EB, teb)
            plsc.subcore_barrier()
            _spmem_writeout(acc, out_hbm, s)

        @pl.when(c == 0)
        def _():
            run(g0_hbm, out0)

        @pl.when(c == 1)
        def _():
            run(g1_hbm, out1)

    return prop


_RB = 1024


def _k1_body(degp_ref, x_ref, w1_ref, g_ref, dinvb_ref):
    deg2d = 1.0 + jnp.sum(degp_ref[:, :, :], axis=0)
    dv = lax.rsqrt(deg2d)
    dv3 = jnp.broadcast_to(dv[:, None, :], (8, 128, 128)).reshape(_RB, 128)
    rowm = lax.broadcasted_iota(jnp.int32, (_RB, 128), 0)
    lanem = lax.broadcasted_iota(jnp.int32, (_RB, 128), 1)
    sel = (lanem == (rowm & 127)).astype(jnp.float32)
    dinv_col = jnp.sum(dv3 * sel, axis=1, keepdims=True)
    hw = jnp.dot(x_ref[:, :], w1_ref[:, :], preferred_element_type=jnp.float32)
    g_ref[:, :] = hw * dinv_col
    dinvb_ref[:, :] = jnp.broadcast_to(dinv_col, (_RB, hw.shape[1]))


def _k2_body(dinvb_ref, sa_ref, sb_ref, b1_ref, w2_ref,
             g0_ref, g1_ref):
    dinvb = dinvb_ref[:, :]
    sfull = sa_ref[:, :] + sb_ref[:, :]
    h1 = jax.nn.relu(sfull * dinvb + b1_ref[0, :][None, :])
    g = jnp.dot(h1, w2_ref[:, :], preferred_element_type=jnp.float32)
    h = g.shape[1] // 2
    g0_ref[:, :] = g[:, :h] * dinvb
    g1_ref[:, :] = g[:, h:] * dinvb


def _k3_body(dinvb_ref, s0_ref, s1_ref, b2_ref, out_ref):
    dinvb = dinvb_ref[:, :]
    sfull = jnp.concatenate([s0_ref[:, :] * dinvb, s1_ref[:, :] * dinvb],
                            axis=1)
    out_ref[:, :] = sfull + b2_ref[0, :][None, :]


def _row_spec(w):
    return pl.BlockSpec((_RB, w), lambda i: (i, 0))


def _degp_spec():
    return pl.BlockSpec((NC * NT, _RB // 128, 128), lambda i: (0, i, 0))


def _full_spec(shape):
    return pl.BlockSpec(shape, lambda i: tuple(0 for _ in shape))


def kernel(x, edge_index, cache_name, W1, b1, W2, b2):
    del cache_name
    n, d_in = x.shape
    d_hid = W1.shape[1]
    d_out = W2.shape[1]
    assert n == N and edge_index.shape == (2, E)
    grid = (-(-n // _RB),)

    src = edge_index[0].astype(jnp.int32)
    dst = edge_index[1].astype(jnp.int32)
    dst_pad = jnp.concatenate([dst, jnp.zeros((16,), jnp.int32)])
    zeros_nh = jnp.zeros((N, d_hid), jnp.float32)

    degp = _make_deg_kernel()(dst_pad, zeros_nh)

    g1, dinvb = pl.pallas_call(
        _k1_body,
        grid=grid,
        in_specs=[_degp_spec(), _row_spec(d_in), _full_spec(W1.shape)],
        out_specs=[_row_spec(d_hid), _row_spec(d_hid)],
        out_shape=[jax.ShapeDtypeStruct((n, d_hid), jnp.float32)] * 2,
    )(degp, x, W1)

    s1a, s1b = _make_prop_edge(d_hid)(g1, zeros_nh, src, dst)

    g20, g21 = pl.pallas_call(
        _k2_body,
        grid=grid,
        in_specs=[_row_spec(d_hid), _row_spec(d_hid), _row_spec(d_hid),
                  _full_spec((1, d_hid)), _full_spec(W2.shape)],
        out_specs=[_row_spec(d_out // 2), _row_spec(d_out // 2)],
        out_shape=[jax.ShapeDtypeStruct((n, d_out // 2), jnp.float32)] * 2,
    )(dinvb, s1a, s1b, b1.reshape(1, d_hid), W2)

    s20, s21 = _make_prop_feat(d_out // 2)(g20, g21, src, dst)

    out = pl.pallas_call(
        _k3_body,
        grid=grid,
        in_specs=[_row_spec(d_out // 2), _row_spec(d_out // 2),
                  _row_spec(d_out // 2), _full_spec((1, d_out))],
        out_specs=_row_spec(d_out),
        out_shape=jax.ShapeDtypeStruct((n, d_out), jnp.float32),
    )(dinvb, s20, s21, b2.reshape(1, d_out))

    return out
